# baseline (speedup 1.0000x reference)
import jax
import jax.numpy as jnp
from jax.experimental import pallas as pl

NQ = 8
K = 1024
D = 64
BLK = 1024


def _rvq_kernel(x_ref, cb_ref, qout_ref, idx_ref, loss_ref, *, n_tokens):
    i = pl.program_id(0)

    @pl.when(i == 0)
    def _init():
        loss_ref[...] = jnp.zeros_like(loss_ref)

    x0 = x_ref[...]
    r = x0
    scale = 1.0 / (n_tokens * D)
    cb_all = cb_ref[...]
    c2_all = jnp.sum(cb_all * cb_all, axis=2)
    idx_cols = []
    losses = []
    for q in range(NQ):
        cb = cb_ref[q]
        dots = jax.lax.dot_general(
            r.astype(jnp.bfloat16), cb.astype(jnp.bfloat16),
            (((1,), (1,)), ((), ())),
            preferred_element_type=jnp.float32)
        c2 = c2_all[q:q + 1, :]
        r2 = jnp.sum(r * r, axis=1, keepdims=True)
        scores = (r2 - 2.0 * dots) + c2
        idx = jnp.argmin(scores, axis=1).astype(jnp.int32)
        oh = (jax.lax.broadcasted_iota(jnp.int32, (BLK, K), 1)
              == idx[:, None]).astype(jnp.float32)
        quant = jax.lax.dot_general(
            oh, cb, (((1,), (0,)), ((), ())),
            precision=jax.lax.Precision.HIGHEST,
            preferred_element_type=jnp.float32)
        r = r - quant
        losses.append(jnp.sum(r * r) * scale)
        idx_cols.append(idx[:, None])
    qout_ref[...] = x0 - r
    idx_ref[...] = jnp.concatenate(idx_cols, axis=1)
    loss_ref[...] += jnp.stack(losses)[None, :]


def kernel(x, codebooks):
    B, N, D_ = x.shape
    n_tokens = B * N
    xf = x.reshape(n_tokens, D_)
    import functools
    body = functools.partial(_rvq_kernel, n_tokens=n_tokens)
    qout, idx, loss = pl.pallas_call(
        body,
        grid=(n_tokens // BLK,),
        in_specs=[
            pl.BlockSpec((BLK, D), lambda i: (i, 0)),
            pl.BlockSpec((NQ, K, D), lambda i: (0, 0, 0)),
        ],
        out_specs=[
            pl.BlockSpec((BLK, D), lambda i: (i, 0)),
            pl.BlockSpec((BLK, NQ), lambda i: (i, 0)),
            pl.BlockSpec((1, NQ), lambda i: (0, 0)),
        ],
        out_shape=[
            jax.ShapeDtypeStruct((n_tokens, D_), jnp.float32),
            jax.ShapeDtypeStruct((n_tokens, NQ), jnp.int32),
            jax.ShapeDtypeStruct((1, NQ), jnp.float32),
        ],
    )(xf, codebooks)
    return (qout.reshape(B, N, D_), idx.reshape(B, N, NQ), loss.reshape(NQ))


# speedup vs baseline: 1.1074x; 1.1074x over previous
import jax
import jax.numpy as jnp
from jax.experimental import pallas as pl

NQ = 8
K = 1024
D = 64
BLK = 2048


def _rvq_kernel(x_ref, cb_ref, qout_ref, idx_ref, loss_ref, *, n_tokens):
    i = pl.program_id(0)

    @pl.when(i == 0)
    def _init():
        loss_ref[...] = jnp.zeros_like(loss_ref)

    x0 = x_ref[...]
    r = x0
    scale = 1.0 / (n_tokens * D)
    idx_cols = []
    losses = []
    for q in range(NQ):
        cb = cb_ref[q]
        dots = jax.lax.dot_general(
            r.astype(jnp.bfloat16), cb.astype(jnp.bfloat16),
            (((1,), (1,)), ((), ())),
            preferred_element_type=jnp.float32)
        c2 = jnp.transpose(jnp.sum(cb * cb, axis=1, keepdims=True))
        r2 = jnp.sum(r * r, axis=1, keepdims=True)
        scores = (r2 - 2.0 * dots) + c2
        idx = jnp.argmin(scores, axis=1).astype(jnp.int32)
        oh = (jax.lax.broadcasted_iota(jnp.int32, (BLK, K), 1)
              == idx[:, None]).astype(jnp.float32)
        quant = jax.lax.dot_general(
            oh, cb, (((1,), (0,)), ((), ())),
            precision=jax.lax.Precision.HIGHEST,
            preferred_element_type=jnp.float32)
        r = r - quant
        losses.append(jnp.sum(r * r) * scale)
        idx_cols.append(idx[:, None])
    qout_ref[...] = x0 - r
    idx_ref[...] = jnp.concatenate(idx_cols, axis=1)
    loss_ref[...] += jnp.stack(losses)[None, :]


def kernel(x, codebooks):
    B, N, D_ = x.shape
    n_tokens = B * N
    xf = x.reshape(n_tokens, D_)
    import functools
    body = functools.partial(_rvq_kernel, n_tokens=n_tokens)
    qout, idx, loss = pl.pallas_call(
        body,
        grid=(n_tokens // BLK,),
        in_specs=[
            pl.BlockSpec((BLK, D), lambda i: (i, 0)),
            pl.BlockSpec((NQ, K, D), lambda i: (0, 0, 0)),
        ],
        out_specs=[
            pl.BlockSpec((BLK, D), lambda i: (i, 0)),
            pl.BlockSpec((BLK, NQ), lambda i: (i, 0)),
            pl.BlockSpec((1, NQ), lambda i: (0, 0)),
        ],
        out_shape=[
            jax.ShapeDtypeStruct((n_tokens, D_), jnp.float32),
            jax.ShapeDtypeStruct((n_tokens, NQ), jnp.int32),
            jax.ShapeDtypeStruct((1, NQ), jnp.float32),
        ],
    )(xf, codebooks)
    return (qout.reshape(B, N, D_), idx.reshape(B, N, NQ), loss.reshape(NQ))


# speedup vs baseline: 1.1622x; 1.0494x over previous
import jax
import jax.numpy as jnp
from jax.experimental import pallas as pl

NQ = 8
K = 1024
D = 64
BLK = 4096


def _rvq_kernel(x_ref, cb_ref, qout_ref, idx_ref, loss_ref, *, n_tokens):
    i = pl.program_id(0)

    @pl.when(i == 0)
    def _init():
        loss_ref[...] = jnp.zeros_like(loss_ref)

    x0 = x_ref[...]
    r = x0
    scale = 1.0 / (n_tokens * D)
    idx_cols = []
    losses = []
    for q in range(NQ):
        cb = cb_ref[q]
        dots = jax.lax.dot_general(
            r.astype(jnp.bfloat16), cb.astype(jnp.bfloat16),
            (((1,), (1,)), ((), ())),
            preferred_element_type=jnp.float32)
        c2 = jnp.transpose(jnp.sum(cb * cb, axis=1, keepdims=True))
        r2 = jnp.sum(r * r, axis=1, keepdims=True)
        scores = (r2 - 2.0 * dots) + c2
        idx = jnp.argmin(scores, axis=1).astype(jnp.int32)
        oh = (jax.lax.broadcasted_iota(jnp.int32, (BLK, K), 1)
              == idx[:, None]).astype(jnp.float32)
        quant = jax.lax.dot_general(
            oh, cb, (((1,), (0,)), ((), ())),
            precision=jax.lax.Precision.HIGHEST,
            preferred_element_type=jnp.float32)
        r = r - quant
        losses.append(jnp.sum(r * r) * scale)
        idx_cols.append(idx[:, None])
    qout_ref[...] = x0 - r
    idx_ref[...] = jnp.concatenate(idx_cols, axis=1)
    loss_ref[...] += jnp.stack(losses)[None, :]


def kernel(x, codebooks):
    B, N, D_ = x.shape
    n_tokens = B * N
    xf = x.reshape(n_tokens, D_)
    import functools
    body = functools.partial(_rvq_kernel, n_tokens=n_tokens)
    qout, idx, loss = pl.pallas_call(
        body,
        grid=(n_tokens // BLK,),
        in_specs=[
            pl.BlockSpec((BLK, D), lambda i: (i, 0)),
            pl.BlockSpec((NQ, K, D), lambda i: (0, 0, 0)),
        ],
        out_specs=[
            pl.BlockSpec((BLK, D), lambda i: (i, 0)),
            pl.BlockSpec((BLK, NQ), lambda i: (i, 0)),
            pl.BlockSpec((1, NQ), lambda i: (0, 0)),
        ],
        out_shape=[
            jax.ShapeDtypeStruct((n_tokens, D_), jnp.float32),
            jax.ShapeDtypeStruct((n_tokens, NQ), jnp.int32),
            jax.ShapeDtypeStruct((1, NQ), jnp.float32),
        ],
    )(xf, codebooks)
    return (qout.reshape(B, N, D_), idx.reshape(B, N, NQ), loss.reshape(NQ))


# speedup vs baseline: 3.2342x; 2.7829x over previous
import jax
import jax.numpy as jnp
from jax.experimental import pallas as pl

NQ = 8
K = 1024
D = 64
BLK = 4096


def _rvq_kernel(x_ref, cb_ref, qout_ref, idx_ref, loss_ref, *, n_tokens):
    i = pl.program_id(0)

    @pl.when(i == 0)
    def _init():
        loss_ref[...] = jnp.zeros_like(loss_ref)

    x0 = x_ref[...]
    r = x0
    scale = 1.0 / (n_tokens * D)
    idx_cols = []
    losses = []
    for q in range(NQ):
        cb = cb_ref[q]
        dots = jax.lax.dot_general(
            r.astype(jnp.bfloat16), cb.astype(jnp.bfloat16),
            (((1,), (1,)), ((), ())),
            preferred_element_type=jnp.float32)
        c2 = jnp.transpose(jnp.sum(cb * cb, axis=1, keepdims=True))
        r2 = jnp.sum(r * r, axis=1, keepdims=True)
        scores = (r2 - 2.0 * dots) + c2
        idx = jnp.argmin(scores, axis=1).astype(jnp.int32)
        oh = (jax.lax.broadcasted_iota(jnp.int32, (BLK, K), 1)
              == idx[:, None]).astype(jnp.bfloat16)
        cb_hi = cb.astype(jnp.bfloat16)
        rem = cb - cb_hi.astype(jnp.float32)
        cb_mid = rem.astype(jnp.bfloat16)
        cb_lo = (rem - cb_mid.astype(jnp.float32)).astype(jnp.bfloat16)
        gain = jnp.concatenate([cb_hi, cb_mid, cb_lo], axis=1)
        q3 = jax.lax.dot_general(
            oh, gain, (((1,), (0,)), ((), ())),
            preferred_element_type=jnp.float32)
        quant = (q3[:, :D] + q3[:, D:2 * D]) + q3[:, 2 * D:]
        r = r - quant
        losses.append(jnp.sum(r * r) * scale)
        idx_cols.append(idx[:, None])
    qout_ref[...] = x0 - r
    idx_ref[...] = jnp.concatenate(idx_cols, axis=1)
    loss_ref[...] += jnp.stack(losses)[None, :]


def kernel(x, codebooks):
    B, N, D_ = x.shape
    n_tokens = B * N
    xf = x.reshape(n_tokens, D_)
    import functools
    body = functools.partial(_rvq_kernel, n_tokens=n_tokens)
    qout, idx, loss = pl.pallas_call(
        body,
        grid=(n_tokens // BLK,),
        in_specs=[
            pl.BlockSpec((BLK, D), lambda i: (i, 0)),
            pl.BlockSpec((NQ, K, D), lambda i: (0, 0, 0)),
        ],
        out_specs=[
            pl.BlockSpec((BLK, D), lambda i: (i, 0)),
            pl.BlockSpec((BLK, NQ), lambda i: (i, 0)),
            pl.BlockSpec((1, NQ), lambda i: (0, 0)),
        ],
        out_shape=[
            jax.ShapeDtypeStruct((n_tokens, D_), jnp.float32),
            jax.ShapeDtypeStruct((n_tokens, NQ), jnp.int32),
            jax.ShapeDtypeStruct((1, NQ), jnp.float32),
        ],
    )(xf, codebooks)
    return (qout.reshape(B, N, D_), idx.reshape(B, N, NQ), loss.reshape(NQ))
